# baseline (speedup 1.0000x reference)
import functools

import jax
import jax.numpy as jnp
from jax import lax
from jax.experimental import pallas as pl
from jax.experimental.pallas import tpu as pltpu
from jax.experimental.pallas import tpu_sc as plsc

N_SPECIES = 1024
N_REACTIONS = 65536
F_SPLIT = 16384
NW = 32
R_TEC = (N_REACTIONS - F_SPLIT) // NW
R_BLOCK = 4096
S_BLOCK = 64
C_BLOCK = 16384


def _sc_rates(scal_ref, ab_ref, al_ref, be_ref, ga_ref, cc_ref, fc_ref,
              rw_ref, sw_ref, rates_ref,
              scal_v, ab_v, al_v, be_v, ga_v, cc_v, fc_v, wv, sv,
              f1_v, f2_v, rt_v, *, w_win, lookback):
    nc = 2
    wid = lax.axis_index("s") * nc + lax.axis_index("c")
    r0 = pl.multiple_of(F_SPLIT + wid * R_TEC, 8)
    start = pl.multiple_of(lax.max(0, 2 * r0 - lookback), 8)

    pltpu.sync_copy(scal_ref, scal_v)
    pltpu.sync_copy(ab_ref, ab_v)
    o0 = pl.multiple_of(wid * R_TEC, 8)
    pltpu.sync_copy(al_ref.at[pl.ds(r0, R_TEC)], al_v)
    pltpu.sync_copy(be_ref.at[pl.ds(r0, R_TEC)], be_v)
    pltpu.sync_copy(ga_ref.at[pl.ds(r0, R_TEC)], ga_v)
    pltpu.sync_copy(cc_ref.at[pl.ds(r0, R_TEC)], cc_v)
    pltpu.sync_copy(fc_ref.at[pl.ds(r0, R_TEC)], fc_v)
    wv[pl.ds(0, 16)] = jnp.full((16,), -1, jnp.int32)
    pltpu.sync_copy(rw_ref.at[pl.ds(start, w_win)], wv.at[pl.ds(8, w_win)])
    pltpu.sync_copy(sw_ref.at[pl.ds(start, w_win)], sv)

    ones = jnp.ones((16,), jnp.float32)

    def init_body(j, _):
        o = pl.multiple_of(j * 16, 16)
        f1_v[pl.ds(o, 16)] = ones
        f2_v[pl.ds(o, 16)] = ones
        return 0

    lax.fori_loop(0, R_TEC // 16, init_body, 0)

    def scatter_body(j, _):
        o = pl.multiple_of(j * 16, 16)
        rwc = wv[pl.ds(8 + o, 16)]
        prev = plsc.load_gather(wv, [lax.iota(jnp.int32, 16) + (7 + o)])
        swc = sv[pl.ds(o, 16)]
        vals = plsc.load_gather(ab_v, [swc])
        ridx = rwc - r0
        in_r = (ridx >= 0) & (ridx < R_TEC)
        first = in_r & (rwc != prev)
        second = in_r & (rwc == prev)
        ridx_c = jnp.clip(ridx, 0, R_TEC - 1)
        plsc.store_scatter(f1_v, [ridx_c], vals, mask=first)
        plsc.store_scatter(f2_v, [ridx_c], vals, mask=second)
        return 0

    lax.fori_loop(0, w_win // 16, scatter_body, 0)

    lt = scal_v[pl.ds(0, 16)]
    invt = scal_v[pl.ds(16, 16)]
    crv = scal_v[pl.ds(32, 16)]
    fuvv = scal_v[pl.ds(48, 16)]

    def rate_body(j, _):
        sl = pl.ds(pl.multiple_of(j * 16, 16), 16)
        r0v = (al_v[sl] * jnp.exp(be_v[sl] * lt - ga_v[sl] * invt)
               + cc_v[sl] * crv + fc_v[sl] * fuvv)
        rt_v[sl] = r0v * f1_v[sl] * f2_v[sl]
        return 0

    lax.fori_loop(0, R_TEC // 16, rate_body, 0)

    pltpu.sync_copy(rt_v, rates_ref.at[pl.ds(o0, R_TEC)])


def _tc_fused(t_ref, cr_ref, fuv_ref, ab_ref, al_ref, be_ref, ga_ref,
              cc_ref, fc_ref, ra_ref, rb_ref, rc_ref, sa_ref, sb_ref,
              sc_ref, inc_ref, out_ref, *, r_block):
    k = pl.program_id(0)
    t = t_ref[0, 0]
    cr = cr_ref[0, 0]
    fuv = fuv_ref[0, 0]
    pb2 = r_block
    w = 3 * pb2

    rates0 = (al_ref[0:1, :] * jnp.exp(be_ref[0:1, :] * jnp.log(t / 300.0)
                                       - ga_ref[0:1, :] / t)
              + cc_ref[0:1, :] * cr + fc_ref[0:1, :] * fuv)

    rw = jnp.concatenate([ra_ref[0:1, :], rb_ref[0:1, :], rc_ref[0:1, :]],
                         axis=1)
    sw = jnp.concatenate([sa_ref[0:1, :], sb_ref[0:1, :], sc_ref[0:1, :]],
                         axis=1)

    la = jnp.log(ab_ref[:, :])
    iota32 = jax.lax.broadcasted_iota(jnp.int32, (32, w), 0)
    oh_lo = jnp.where(iota32 == (sw & 31), 1.0, 0.0)
    cols = jax.lax.dot_general(la, oh_lo, (((1,), (0,)), ((), ())),
                               preferred_element_type=jnp.float32)
    f = jnp.sum(jnp.where(iota32 == (sw >> 5), cols, 0.0),
                axis=0, keepdims=True)

    pos = jax.lax.broadcasted_iota(jnp.int32, (1, w), 1)
    v = jnp.where((k > 0) | (pos >= pb2), f, 0.0)

    off = rw - k * r_block
    hi_rows = r_block >> 5
    iota_hi = jax.lax.broadcasted_iota(jnp.int32, (hi_rows, w), 0)
    bv = jnp.where(iota_hi == (off >> 5), v, 0.0)
    oh_lo2 = jnp.where(iota32 == (off & 31), 1.0, 0.0)
    g = jax.lax.dot_general(bv, oh_lo2, (((1,), (1,)), ((), ())),
                            preferred_element_type=jnp.float32)

    e = jnp.exp(g)
    tiled = jnp.tile(e, (1, hi_rows))
    lane = jax.lax.broadcasted_iota(jnp.int32, (hi_rows, r_block), 1)
    rows = jax.lax.broadcasted_iota(jnp.int32, (hi_rows, r_block), 0)
    flat = jnp.sum(jnp.where(rows == (lane >> 5), tiled, 0.0),
                   axis=0, keepdims=True)

    rates = rates0 * flat

    contrib = jax.lax.dot_general(inc_ref[:, :], rates,
                                  (((1,), (1,)), ((), ())),
                                  preferred_element_type=jnp.float32)

    @pl.when(k == 0)
    def _init():
        out_ref[:, :] = contrib

    @pl.when(k > 0)
    def _acc():
        out_ref[:, :] += contrib


def _tc_matvec2(yp_ref, rates_ref, inc_ref, out_ref):
    c = pl.program_id(1)
    contrib = jax.lax.dot_general(
        inc_ref[:, :], rates_ref[0:1, :], (((1,), (1,)), ((), ())),
        preferred_element_type=jnp.float32)

    @pl.when(c == 0)
    def _init():
        out_ref[:, :] = yp_ref[:, :] + contrib

    @pl.when(c > 0)
    def _acc():
        out_ref[:, :] += contrib


def kernel(abundances, temperature, cr_rate, fuv_rate, incidence, alpha, beta,
           gamma, cr_coef, fuv_coef, reac_idx, species_idx):
    n_pairs = reac_idx.shape[0]
    deficit = 2 * N_REACTIONS - n_pairs
    lookback = -(-deficit // 8) * 8
    w_win = 2 * R_TEC + -(-lookback // 16) * 16
    if lookback > R_BLOCK or lookback > 2 * R_TEC:
        raise ValueError("pair-list deficit exceeds a pair window")

    l_need = max(2 * (F_SPLIT + (NW - 1) * R_TEC) - lookback + w_win,
                 2 * N_REACTIONS)
    l_pad = -(-l_need // R_BLOCK) * R_BLOCK
    pad = l_pad - n_pairs
    rw = jnp.pad(reac_idx.astype(jnp.int32), (0, pad),
                 constant_values=N_REACTIONS)
    sw = jnp.pad(species_idx.astype(jnp.int32), (0, pad), constant_values=0)

    t = temperature.astype(jnp.float32)
    scal = jnp.concatenate([
        jnp.full((16,), jnp.log(t / 300.0), jnp.float32),
        jnp.full((16,), 1.0 / t, jnp.float32),
        jnp.full((16,), cr_rate.astype(jnp.float32), jnp.float32),
        jnp.full((16,), fuv_rate.astype(jnp.float32), jnp.float32),
    ])

    sc_rates = functools.partial(
        pl.kernel,
        mesh=plsc.VectorSubcoreMesh(core_axis_name="c", subcore_axis_name="s"),
        compiler_params=pltpu.CompilerParams(needs_layout_passes=False),
        out_type=jax.ShapeDtypeStruct((N_REACTIONS - F_SPLIT,), jnp.float32),
        scratch_types=[
            pltpu.VMEM((64,), jnp.float32),
            pltpu.VMEM((N_SPECIES,), jnp.float32),
            pltpu.VMEM((R_TEC,), jnp.float32),
            pltpu.VMEM((R_TEC,), jnp.float32),
            pltpu.VMEM((R_TEC,), jnp.float32),
            pltpu.VMEM((R_TEC,), jnp.float32),
            pltpu.VMEM((R_TEC,), jnp.float32),
            pltpu.VMEM((8 + w_win,), jnp.int32),
            pltpu.VMEM((w_win,), jnp.int32),
            pltpu.VMEM((R_TEC,), jnp.float32),
            pltpu.VMEM((R_TEC,), jnp.float32),
            pltpu.VMEM((R_TEC,), jnp.float32),
        ],
    )(functools.partial(_sc_rates, w_win=w_win, lookback=lookback))

    rates_rest = sc_rates(scal, abundances, alpha, beta, gamma, cr_coef,
                          fuv_coef, rw, sw)

    r = R_BLOCK
    nb1 = F_SPLIT // r
    rw2 = rw.reshape(1, l_pad)
    sw2 = sw.reshape(1, l_pad)
    row = lambda x: x.reshape(1, -1)
    scl = lambda x: x.reshape(1, 1).astype(jnp.float32)
    pair_a = pl.BlockSpec((1, r), lambda k: (0, jnp.maximum(2 * k - 1, 0)))
    pair_b = pl.BlockSpec((1, r), lambda k: (0, 2 * k))
    pair_c = pl.BlockSpec((1, r), lambda k: (0, 2 * k + 1))
    param = pl.BlockSpec((1, r), lambda k: (0, k))
    whole = lambda shape: pl.BlockSpec(shape, lambda k: (0, 0))

    y_part = pl.pallas_call(
        functools.partial(_tc_fused, r_block=r),
        grid=(nb1,),
        in_specs=[
            whole((1, 1)), whole((1, 1)), whole((1, 1)),
            whole((32, 32)),
            param, param, param, param, param,
            pair_a, pair_b, pair_c, pair_a, pair_b, pair_c,
            pl.BlockSpec((N_SPECIES, r), lambda k: (0, k)),
        ],
        out_specs=pl.BlockSpec((N_SPECIES, 1), lambda k: (0, 0)),
        out_shape=jax.ShapeDtypeStruct((N_SPECIES, 1), jnp.float32),
        compiler_params=pltpu.CompilerParams(
            dimension_semantics=("arbitrary",),
        ),
    )(scl(temperature), scl(cr_rate), scl(fuv_rate),
      abundances.reshape(32, 32),
      row(alpha), row(beta), row(gamma), row(cr_coef), row(fuv_coef),
      rw2, rw2, rw2, sw2, sw2, sw2, incidence)

    nc2 = (N_REACTIONS - F_SPLIT) // C_BLOCK
    off_c = F_SPLIT // C_BLOCK
    out = pl.pallas_call(
        _tc_matvec2,
        grid=(N_SPECIES // S_BLOCK, nc2),
        in_specs=[
            pl.BlockSpec((S_BLOCK, 1), lambda k, c: (k, 0)),
            pl.BlockSpec((1, C_BLOCK), lambda k, c: (0, c)),
            pl.BlockSpec((S_BLOCK, C_BLOCK), lambda k, c: (k, c + off_c)),
        ],
        out_specs=pl.BlockSpec((S_BLOCK, 1), lambda k, c: (k, 0)),
        out_shape=jax.ShapeDtypeStruct((N_SPECIES, 1), jnp.float32),
        compiler_params=pltpu.CompilerParams(
            dimension_semantics=("arbitrary", "arbitrary"),
        ),
    )(y_part, rates_rest.reshape(1, N_REACTIONS - F_SPLIT), incidence)
    return out.reshape(N_SPECIES)


# speedup vs baseline: 1.0273x; 1.0273x over previous
import functools

import jax
import jax.numpy as jnp
from jax import lax
from jax.experimental import pallas as pl
from jax.experimental.pallas import tpu as pltpu
from jax.experimental.pallas import tpu_sc as plsc

N_SPECIES = 1024
N_REACTIONS = 65536
S_BLOCK = 64
NW = 32
R_TEC = N_REACTIONS // NW


def _sc_rates(scal_ref, ab_ref, al_ref, be_ref, ga_ref, cc_ref, fc_ref,
              rw_ref, sw_ref, rates_ref,
              scal_v, ab_v, al_v, be_v, ga_v, cc_v, fc_v, wv, sv,
              f1_v, f2_v, rt_v, *, w_win, lookback):
    nc = 2
    wid = lax.axis_index("s") * nc + lax.axis_index("c")
    r0 = pl.multiple_of(wid * R_TEC, 8)
    start = pl.multiple_of(lax.max(0, 4096 * wid - lookback), 8)

    pltpu.sync_copy(scal_ref, scal_v)
    pltpu.sync_copy(ab_ref, ab_v)
    pltpu.sync_copy(al_ref.at[pl.ds(r0, R_TEC)], al_v)
    pltpu.sync_copy(be_ref.at[pl.ds(r0, R_TEC)], be_v)
    pltpu.sync_copy(ga_ref.at[pl.ds(r0, R_TEC)], ga_v)
    pltpu.sync_copy(cc_ref.at[pl.ds(r0, R_TEC)], cc_v)
    pltpu.sync_copy(fc_ref.at[pl.ds(r0, R_TEC)], fc_v)
    wv[pl.ds(0, 16)] = jnp.full((16,), -1, jnp.int32)
    pltpu.sync_copy(rw_ref.at[pl.ds(start, w_win)], wv.at[pl.ds(8, w_win)])
    pltpu.sync_copy(sw_ref.at[pl.ds(start, w_win)], sv)


    lt = scal_v[pl.ds(0, 16)]
    invt = scal_v[pl.ds(16, 16)]
    crv = scal_v[pl.ds(32, 16)]
    fuvv = scal_v[pl.ds(48, 16)]

    ones = jnp.ones((16,), jnp.float32)

    def init_body(j, _):
        for u in range(4):
            o = pl.multiple_of(j * 64 + u * 16, 16)
            f1_v[pl.ds(o, 16)] = ones
            f2_v[pl.ds(o, 16)] = ones
        return 0

    lax.fori_loop(0, R_TEC // 64, init_body, 0)

    def scatter_body(j, _):
        for u in range(4):
            o = pl.multiple_of(j * 64 + u * 16, 16)
            rwc = wv[pl.ds(8 + o, 16)]
            prev = plsc.load_gather(wv, [lax.iota(jnp.int32, 16) + (7 + o)])
            swc = sv[pl.ds(o, 16)]
            vals = plsc.load_gather(ab_v, [swc])
            ridx = rwc - r0
            in_r = (ridx >= 0) & (ridx < R_TEC)
            first = in_r & (rwc != prev)
            second = in_r & (rwc == prev)
            ridx_c = jnp.clip(ridx, 0, R_TEC - 1)
            plsc.store_scatter(f1_v, [ridx_c], vals, mask=first)
            plsc.store_scatter(f2_v, [ridx_c], vals, mask=second)
        return 0

    lax.fori_loop(0, w_win // 64, scatter_body, 0)

    def rate_body(j, _):
        for u in range(4):
            sl = pl.ds(pl.multiple_of(j * 64 + u * 16, 16), 16)
            r0v = (al_v[sl] * jnp.exp(be_v[sl] * lt - ga_v[sl] * invt)
                   + cc_v[sl] * crv + fc_v[sl] * fuvv)
            rt_v[sl] = r0v * f1_v[sl] * f2_v[sl]
        return 0

    lax.fori_loop(0, R_TEC // 64, rate_body, 0)

    pltpu.sync_copy(rt_v, rates_ref.at[pl.ds(r0, R_TEC)])


def _tc_matvec(rates_ref, inc_ref, out_ref):
    out_ref[:, :] = jax.lax.dot_general(
        inc_ref[:, :], rates_ref[0:1, :], (((1,), (1,)), ((), ())),
        preferred_element_type=jnp.float32)


def kernel(abundances, temperature, cr_rate, fuv_rate, incidence, alpha, beta,
           gamma, cr_coef, fuv_coef, reac_idx, species_idx):
    n_pairs = reac_idx.shape[0]
    deficit = 2 * N_REACTIONS - n_pairs
    lookback = -(-deficit // 8) * 8
    w_win = 2 * R_TEC + -(-lookback // 16) * 16
    if lookback > 2 * R_TEC:
        raise ValueError("pair-list deficit exceeds one subcore pair range")

    l_pad = 4096 * (NW - 1) - lookback + w_win
    pad = l_pad - n_pairs
    rw = jnp.pad(reac_idx.astype(jnp.int32), (0, pad),
                 constant_values=N_REACTIONS)
    sw = jnp.pad(species_idx.astype(jnp.int32), (0, pad), constant_values=0)

    t = temperature.astype(jnp.float32)
    scal = jnp.concatenate([
        jnp.full((16,), jnp.log(t / 300.0), jnp.float32),
        jnp.full((16,), 1.0 / t, jnp.float32),
        jnp.full((16,), cr_rate.astype(jnp.float32), jnp.float32),
        jnp.full((16,), fuv_rate.astype(jnp.float32), jnp.float32),
    ])

    sc_rates = functools.partial(
        pl.kernel,
        mesh=plsc.VectorSubcoreMesh(core_axis_name="c", subcore_axis_name="s"),
        compiler_params=pltpu.CompilerParams(needs_layout_passes=False),
        out_type=jax.ShapeDtypeStruct((N_REACTIONS,), jnp.float32),
        scratch_types=[
            pltpu.VMEM((64,), jnp.float32),
            pltpu.VMEM((N_SPECIES,), jnp.float32),
            pltpu.VMEM((R_TEC,), jnp.float32),
            pltpu.VMEM((R_TEC,), jnp.float32),
            pltpu.VMEM((R_TEC,), jnp.float32),
            pltpu.VMEM((R_TEC,), jnp.float32),
            pltpu.VMEM((R_TEC,), jnp.float32),
            pltpu.VMEM((8 + w_win,), jnp.int32),
            pltpu.VMEM((w_win,), jnp.int32),
            pltpu.VMEM((R_TEC,), jnp.float32),
            pltpu.VMEM((R_TEC,), jnp.float32),
            pltpu.VMEM((R_TEC,), jnp.float32),
        ],
    )(functools.partial(_sc_rates, w_win=w_win, lookback=lookback))

    rates = sc_rates(scal, abundances, alpha, beta, gamma, cr_coef, fuv_coef,
                     rw, sw)

    out = pl.pallas_call(
        _tc_matvec,
        grid=(N_SPECIES // S_BLOCK,),
        in_specs=[
            pl.BlockSpec((1, N_REACTIONS), lambda k: (0, 0)),
            pl.BlockSpec((S_BLOCK, N_REACTIONS), lambda k: (k, 0)),
        ],
        out_specs=pl.BlockSpec((S_BLOCK, 1), lambda k: (k, 0)),
        out_shape=jax.ShapeDtypeStruct((N_SPECIES, 1), jnp.float32),
        compiler_params=pltpu.CompilerParams(
            dimension_semantics=("arbitrary",),
        ),
    )(rates.reshape(1, N_REACTIONS), incidence)
    return out.reshape(N_SPECIES)
